# Initial kernel scaffold; baseline (speedup 1.0000x reference)
import functools

import jax
import jax.numpy as jnp
from jax import lax
from jax.experimental import pallas as pl
from jax.experimental.pallas import tpu as pltpu
from jax.experimental.pallas import tpu_sc as plsc

B, L, V, D = 4096, 200, 100000, 64

NC, NS = 2, 16
NW = NC * NS
BPW = B // NW
NG = BPW // 16

ROW_BLK = 4000


def _score_body(emb_ref, w_ref, b_ref, out_ref):
    e = emb_ref[...]
    w = w_ref[...]
    out_ref[...] = jnp.sum(e * w, axis=1, keepdims=True) + b_ref[0]


def _scores(emb, W, b):
    return pl.pallas_call(
        _score_body,
        grid=(V // ROW_BLK,),
        in_specs=[
            pl.BlockSpec((ROW_BLK, D), lambda i: (i, 0)),
            pl.BlockSpec((1, D), lambda i: (0, 0)),
            pl.BlockSpec(memory_space=pltpu.SMEM),
        ],
        out_specs=pl.BlockSpec((ROW_BLK, 1), lambda i: (i, 0)),
        out_shape=jax.ShapeDtypeStruct((V, 1), jnp.float32),
    )(emb, W, b)


def _pool_body(s_hbm, xt_hbm, out_hbm, s_v, x_a, x_b, o_v, sem_s, sem_a, sem_b):
    wid = lax.axis_index("s") * NC + lax.axis_index("c")

    cp_s = pltpu.make_async_copy(s_hbm, s_v, sem_s)
    cp_s.start()
    x_bufs = (x_a, x_b)
    x_sems = (sem_a, sem_b)
    pltpu.make_async_copy(xt_hbm.at[wid, 0], x_a, sem_a).start()
    cp_s.wait()

    inv_l = jnp.float32(1.0 / L)

    for g in range(NG):
        buf = x_bufs[g % 2]
        sem = x_sems[g % 2]
        pltpu.make_async_copy(xt_hbm.at[wid, g], buf, sem).wait()
        if g + 1 < NG:
            nbuf = x_bufs[(g + 1) % 2]
            nsem = x_sems[(g + 1) % 2]
            pltpu.make_async_copy(xt_hbm.at[wid, g + 1], nbuf, nsem).start()

        def step(l, acc, buf=buf):
            idx = buf[l]
            return acc + plsc.load_gather(s_v, [idx])

        acc = lax.fori_loop(0, L, step, jnp.zeros((16,), jnp.float32))
        z = acc * inv_l
        o_v[pl.ds(g * 16, 16)] = 1.0 / (1.0 + jnp.exp(-z))

    pltpu.sync_copy(o_v, out_hbm.at[pl.ds(wid * BPW, BPW)])


_pool = functools.partial(
    pl.kernel,
    out_type=jax.ShapeDtypeStruct((B,), jnp.float32),
    mesh=plsc.VectorSubcoreMesh(core_axis_name="c", subcore_axis_name="s"),
    scratch_types=[
        pltpu.VMEM((V,), jnp.float32),
        pltpu.VMEM((L, 16), jnp.int32),
        pltpu.VMEM((L, 16), jnp.int32),
        pltpu.VMEM((BPW,), jnp.float32),
        pltpu.SemaphoreType.DMA,
        pltpu.SemaphoreType.DMA,
        pltpu.SemaphoreType.DMA,
    ],
)(_pool_body)


@jax.jit
def kernel(x, emb, W, b):
    s = _scores(emb, W, b).reshape(V)
    xt = (
        x.astype(jnp.int32)
        .reshape(NW, NG, 16, L)
        .transpose(0, 1, 3, 2)
    )
    return _pool(s, xt)


# speedup vs baseline: 19.6099x; 19.6099x over previous
import functools

import jax
import jax.numpy as jnp
from jax import lax
from jax.experimental import pallas as pl
from jax.experimental.pallas import tpu as pltpu
from jax.experimental.pallas import tpu_sc as plsc

B, L, V, D = 4096, 200, 100000, 64

NC, NS = 2, 16
NW = NC * NS
BPW = B // NW
NG = BPW // 16

ROW_BLK = 4000


def _score_body(emb_ref, w_ref, b_ref, out_ref):
    e = emb_ref[...]
    w = w_ref[...]
    out_ref[...] = jnp.sum(e * w, axis=1, keepdims=True) + b_ref[0]


def _scores(emb, W, b):
    return pl.pallas_call(
        _score_body,
        grid=(V // ROW_BLK,),
        in_specs=[
            pl.BlockSpec((ROW_BLK, D), lambda i: (i, 0)),
            pl.BlockSpec((1, D), lambda i: (0, 0)),
            pl.BlockSpec(memory_space=pltpu.SMEM),
        ],
        out_specs=pl.BlockSpec((ROW_BLK, 1), lambda i: (i, 0)),
        out_shape=jax.ShapeDtypeStruct((V, 1), jnp.float32),
    )(emb, W, b)


def _pool_body(s_hbm, xt_hbm, out_hbm, s_v, x_a, x_b, o_v, sem_s, sem_a, sem_b):
    wid = lax.axis_index("s") * NC + lax.axis_index("c")

    gbase = wid * NG

    cp_s = pltpu.make_async_copy(s_hbm, s_v, sem_s)
    cp_s.start()
    x_bufs = (x_a, x_b)
    x_sems = (sem_a, sem_b)
    pltpu.make_async_copy(xt_hbm.at[gbase], x_a, sem_a).start()
    cp_s.wait()

    inv_l = jnp.float32(1.0 / L)

    for g in range(NG):
        buf = x_bufs[g % 2]
        sem = x_sems[g % 2]
        pltpu.make_async_copy(xt_hbm.at[gbase + g], buf, sem).wait()
        if g + 1 < NG:
            nbuf = x_bufs[(g + 1) % 2]
            nsem = x_sems[(g + 1) % 2]
            pltpu.make_async_copy(xt_hbm.at[gbase + g + 1], nbuf, nsem).start()

        def step(l, acc, buf=buf):
            idx = buf[pl.ds(l * 16, 16)]
            return acc + plsc.load_gather(s_v, [idx])

        acc = lax.fori_loop(0, L, step, jnp.zeros((16,), jnp.float32))
        z = acc * inv_l
        o_v[pl.ds(g * 16, 16)] = 1.0 / (1.0 + jnp.exp(-z))

    pltpu.sync_copy(o_v, out_hbm.at[pl.ds(wid * BPW, BPW)])


_pool = functools.partial(
    pl.kernel,
    out_type=jax.ShapeDtypeStruct((B,), jnp.float32),
    mesh=plsc.VectorSubcoreMesh(core_axis_name="c", subcore_axis_name="s"),
    compiler_params=pltpu.CompilerParams(needs_layout_passes=False),
    scratch_types=[
        pltpu.VMEM((V,), jnp.float32),
        pltpu.VMEM((L * 16,), jnp.int32),
        pltpu.VMEM((L * 16,), jnp.int32),
        pltpu.VMEM((BPW,), jnp.float32),
        pltpu.SemaphoreType.DMA,
        pltpu.SemaphoreType.DMA,
        pltpu.SemaphoreType.DMA,
    ],
)(_pool_body)


@jax.jit
def kernel(x, emb, W, b):
    s = _scores(emb, W, b).reshape(V)
    xt = (
        x.astype(jnp.int32)
        .reshape(NW * NG, 16, L)
        .transpose(0, 2, 1)
        .reshape(NW * NG, L * 16)
    )
    return _pool(s, xt)


# speedup vs baseline: 53.1119x; 2.7084x over previous
import functools

import jax
import jax.numpy as jnp
from jax import lax
from jax.experimental import pallas as pl
from jax.experimental.pallas import tpu as pltpu
from jax.experimental.pallas import tpu_sc as plsc

B, L, V, D = 4096, 200, 100000, 64

NC, NS = 2, 16
NW = NC * NS
BPW = B // NW
NG = BPW // 16

COL_BLK = 4096


def _score_body(embt_ref, wt_ref, b_ref, out_ref):
    e = embt_ref[...]
    w = wt_ref[...]
    out_ref[...] = jnp.sum(e * w, axis=0) + b_ref[0]


def _scores(embt, wt, b):
    grid = (V + COL_BLK - 1) // COL_BLK
    return pl.pallas_call(
        _score_body,
        grid=(grid,),
        in_specs=[
            pl.BlockSpec((D, COL_BLK), lambda i: (0, i)),
            pl.BlockSpec((D, 1), lambda i: (0, 0)),
            pl.BlockSpec(memory_space=pltpu.SMEM),
        ],
        out_specs=pl.BlockSpec((COL_BLK,), lambda i: (i,)),
        out_shape=jax.ShapeDtypeStruct((V,), jnp.float32),
    )(embt, wt, b)


def _pool_body(s_hbm, xt_hbm, out_hbm, s_v, x_v, o_v, sem_s, sem_x):
    wid = lax.axis_index("s") * NC + lax.axis_index("c")
    base = wid * BPW

    cp_s = pltpu.make_async_copy(s_hbm, s_v, sem_s)
    cp_s.start()
    cp_x = pltpu.make_async_copy(xt_hbm.at[:, pl.ds(base, BPW)], x_v, sem_x)
    cp_x.start()
    cp_s.wait()
    cp_x.wait()

    zero = jnp.zeros((16,), jnp.float32)

    def step(l, accs):
        new = []
        for g in range(NG):
            idx = x_v[l, pl.ds(g * 16, 16)]
            new.append(accs[g] + plsc.load_gather(s_v, [idx]))
        return tuple(new)

    accs = lax.fori_loop(0, L, step, (zero,) * NG)

    inv_l = jnp.float32(1.0 / L)
    for g in range(NG):
        z = accs[g] * inv_l
        o_v[pl.ds(g * 16, 16)] = 1.0 / (1.0 + jnp.exp(-z))

    pltpu.sync_copy(o_v, out_hbm.at[pl.ds(base, BPW)])


_pool = functools.partial(
    pl.kernel,
    out_type=jax.ShapeDtypeStruct((B,), jnp.float32),
    mesh=plsc.VectorSubcoreMesh(core_axis_name="c", subcore_axis_name="s"),
    compiler_params=pltpu.CompilerParams(needs_layout_passes=False),
    scratch_types=[
        pltpu.VMEM((V,), jnp.float32),
        pltpu.VMEM((L, BPW), jnp.int32),
        pltpu.VMEM((BPW,), jnp.float32),
        pltpu.SemaphoreType.DMA,
        pltpu.SemaphoreType.DMA,
    ],
)(_pool_body)


@jax.jit
def kernel(x, emb, W, b):
    s = _scores(emb.T, W.T, b)
    return _pool(s, x.astype(jnp.int32).T)


# speedup vs baseline: 59.2811x; 1.1162x over previous
import functools

import jax
import jax.numpy as jnp
from jax import lax
from jax.experimental import pallas as pl
from jax.experimental.pallas import tpu as pltpu
from jax.experimental.pallas import tpu_sc as plsc

B, L, V, D = 4096, 200, 100000, 64

NC, NS = 2, 16
NW = NC * NS
BPW = B // NW
NG = BPW // 16

COL_BLK = 8192


def _score_body(embt_ref, wt_ref, b_ref, out_ref):
    e = embt_ref[...]
    w = wt_ref[...]
    out_ref[...] = jnp.sum(e * w, axis=0) + b_ref[0]


def _scores(embt, wt, b):
    grid = (V + COL_BLK - 1) // COL_BLK
    return pl.pallas_call(
        _score_body,
        grid=(grid,),
        in_specs=[
            pl.BlockSpec((D, COL_BLK), lambda i: (0, i)),
            pl.BlockSpec((D, 1), lambda i: (0, 0)),
            pl.BlockSpec(memory_space=pltpu.SMEM),
        ],
        out_specs=pl.BlockSpec((COL_BLK,), lambda i: (i,)),
        out_shape=jax.ShapeDtypeStruct((V,), jnp.float32),
    )(embt, wt, b)


def _pool_body(s_hbm, xt_hbm, out_hbm, s_v, x_v, o_v, sem_s, sem_x):
    wid = lax.axis_index("s") * NC + lax.axis_index("c")
    base = wid * BPW

    cp_s = pltpu.make_async_copy(s_hbm, s_v, sem_s)
    cp_s.start()
    cp_x = pltpu.make_async_copy(xt_hbm.at[:, pl.ds(base, BPW)], x_v, sem_x)
    cp_x.start()
    cp_s.wait()
    cp_x.wait()

    zero = jnp.zeros((16,), jnp.float32)

    @plsc.parallel_loop(0, L, unroll=4, carry=(zero,) * NG)
    def accs(l, accs):
        new = []
        for g in range(NG):
            idx = x_v[l, pl.ds(g * 16, 16)]
            new.append(accs[g] + plsc.load_gather(s_v, [idx]))
        return tuple(new)

    inv_l = jnp.float32(1.0 / L)
    for g in range(NG):
        z = accs[g] * inv_l
        o_v[pl.ds(g * 16, 16)] = 1.0 / (1.0 + jnp.exp(-z))

    pltpu.sync_copy(o_v, out_hbm.at[pl.ds(base, BPW)])


_pool = functools.partial(
    pl.kernel,
    out_type=jax.ShapeDtypeStruct((B,), jnp.float32),
    mesh=plsc.VectorSubcoreMesh(core_axis_name="c", subcore_axis_name="s"),
    compiler_params=pltpu.CompilerParams(needs_layout_passes=False),
    scratch_types=[
        pltpu.VMEM((V,), jnp.float32),
        pltpu.VMEM((L, BPW), jnp.int32),
        pltpu.VMEM((BPW,), jnp.float32),
        pltpu.SemaphoreType.DMA,
        pltpu.SemaphoreType.DMA,
    ],
)(_pool_body)


@jax.jit
def kernel(x, emb, W, b):
    s = _scores(emb.T, W.T, b)
    return _pool(s, x.astype(jnp.int32).T)
